# baseline (speedup 1.0000x reference)
import functools
import jax
import jax.numpy as jnp
from jax import lax
from jax.experimental import pallas as pl
from jax.experimental.pallas import tpu as pltpu
from jax.experimental.pallas import tpu_sc as plsc

N_FIELDS = 26
VOCAB = 100000
D = 32
H = 64
GROUPS = [(0, 13), (13, 13)]


def _make_sc_gather(nf, batch):
    info = plsc.get_sparse_core_info()
    nc, ns = info.num_cores, info.num_subcores
    nw = nc * ns
    rows = nf * D
    rows_per_w = rows // nw
    mesh = plsc.VectorSubcoreMesh(core_axis_name="c", subcore_axis_name="s")

    @functools.partial(
        pl.kernel,
        mesh=mesh,
        compiler_params=pltpu.CompilerParams(use_tc_tiling_on_sc=False),
        out_type=jax.ShapeDtypeStruct((rows, batch), jnp.float32),
        scratch_types=[
            pltpu.VMEM((batch,), jnp.int32),
            pltpu.VMEM((2, batch), jnp.int32),
            pltpu.VMEM((2, batch), jnp.float32),
            pltpu.SemaphoreType.DMA,
            pltpu.SemaphoreType.DMA,
        ],
    )
    def gather_k(table_hbm, xt_hbm, out_hbm, xf_v, idx_v, row_v, sg, so):
        wid = lax.axis_index("s") * nc + lax.axis_index("c")
        p0 = wid * rows_per_w

        def build(i):
            p = p0 + i
            pltpu.sync_copy(xt_hbm.at[p // D], xf_v)
            base = p * VOCAB

            def addv(j, carry):
                idx_v[i % 2, pl.ds(j * 16, 16)] = (
                    xf_v[pl.ds(j * 16, 16)] + base)
                return carry

            lax.fori_loop(0, batch // 16, addv, 0, unroll=False)

        def gather(i):
            return pltpu.make_async_copy(
                table_hbm.at[idx_v.at[i % 2]], row_v.at[i % 2], sg)

        def writeout(i):
            return pltpu.make_async_copy(
                row_v.at[i % 2], out_hbm.at[p0 + i], so)

        build(0)
        for i in range(rows_per_w):
            if i >= 2:
                writeout(i - 2).wait()
            gather(i).start()
            if i + 1 < rows_per_w:
                build(i + 1)
            gather(i).wait()
            writeout(i).start()
        writeout(rows_per_w - 2).wait()
        writeout(rows_per_w - 1).wait()

    return gather_k


def _mlp_body(e0, e1, w0, w1, b1_ref, w2o_ref, b2_ref, o_ref):
    ht = jnp.dot(w0[...], e0[...], preferred_element_type=jnp.float32)
    ht += jnp.dot(w1[...], e1[...], preferred_element_type=jnp.float32)
    ht = jnp.maximum(ht + b1_ref[...], 0.0)
    o_ref[...] = jnp.dot(w2o_ref[...], ht,
                         preferred_element_type=jnp.float32) + b2_ref[...]


def kernel(x, tables, W1, b1, W2, b2):
    batch = x.shape[0]
    tt = jnp.transpose(tables, (0, 2, 1))
    xt = jnp.transpose(x.astype(jnp.int32), (1, 0))
    w1t = jnp.transpose(W1, (1, 0))

    eparts, wparts = [], []
    for f0, nf in GROUPS:
        tlin_g = tt[f0:f0 + nf].reshape(nf * D * VOCAB)
        e_g = _make_sc_gather(nf, batch)(tlin_g, xt[f0:f0 + nf])
        eparts.append(e_g)
        wparts.append(w1t[:, f0 * D:(f0 + nf) * D])

    blk = 1024
    e_specs = [pl.BlockSpec((nf * D, blk), lambda i: (0, i))
               for _, nf in GROUPS]
    w_specs = [pl.BlockSpec((H, nf * D), lambda i: (0, 0))
               for _, nf in GROUPS]
    outT = pl.pallas_call(
        _mlp_body,
        grid=(batch // blk,),
        in_specs=e_specs + w_specs + [
            pl.BlockSpec((H, 1), lambda i: (0, 0)),
            pl.BlockSpec((1, H), lambda i: (0, 0)),
            pl.BlockSpec((1, 1), lambda i: (0, 0)),
        ],
        out_specs=pl.BlockSpec((1, blk), lambda i: (0, i)),
        out_shape=jax.ShapeDtypeStruct((1, batch), jnp.float32),
    )(*eparts, *wparts, b1.reshape(H, 1), W2.reshape(1, H), b2.reshape(1, 1))
    return outT.reshape(batch, 1)


# speedup vs baseline: 1.2304x; 1.2304x over previous
import functools
import jax
import jax.numpy as jnp
from jax import lax
from jax.experimental import pallas as pl
from jax.experimental.pallas import tpu as pltpu
from jax.experimental.pallas import tpu_sc as plsc

N_FIELDS = 26
VOCAB = 100000
D = 32
H = 64
FD = N_FIELDS * D


def _make_sc_gather(batch):
    info = plsc.get_sparse_core_info()
    nc, ns = info.num_cores, info.num_subcores
    nw = nc * ns
    rows_per_w = FD // nw
    mesh = plsc.VectorSubcoreMesh(core_axis_name="c", subcore_axis_name="s")

    @functools.partial(
        pl.kernel,
        mesh=mesh,
        compiler_params=pltpu.CompilerParams(use_tc_tiling_on_sc=False),
        out_type=jax.ShapeDtypeStruct((FD, batch), jnp.float32),
        scratch_types=[
            pltpu.VMEM((2, batch), jnp.int32),
            pltpu.VMEM((2, batch), jnp.int32),
            pltpu.VMEM((2, batch), jnp.float32),
            pltpu.SemaphoreType.DMA,
            pltpu.SemaphoreType.DMA,
        ],
    )
    def gather_k(table_hbm, xt_hbm, out_hbm, xf_v, idx_v, row_v, sg, so):
        wid = lax.axis_index("s") * nc + lax.axis_index("c")
        p0 = wid * rows_per_w
        fstart = jnp.minimum(p0 // D, N_FIELDS - 2)
        pltpu.sync_copy(xt_hbm.at[pl.ds(fstart, 2)], xf_v)

        def build(i):
            p = p0 + i
            frel = p // D - fstart
            base = p * VOCAB

            def addv(j, carry):
                idx_v[i % 2, pl.ds(j * 16, 16)] = (
                    xf_v[frel, pl.ds(j * 16, 16)] + base)
                return carry

            lax.fori_loop(0, batch // 16, addv, 0, unroll=False)

        def gather(i):
            return pltpu.make_async_copy(
                table_hbm.at[idx_v.at[i % 2]], row_v.at[i % 2], sg)

        def writeout(i):
            return pltpu.make_async_copy(
                row_v.at[i % 2], out_hbm.at[p0 + i], so)

        build(0)
        for i in range(rows_per_w):
            if i >= 2:
                writeout(i - 2).wait()
            gather(i).start()
            if i + 1 < rows_per_w:
                build(i + 1)
            gather(i).wait()
            writeout(i).start()
        writeout(rows_per_w - 2).wait()
        writeout(rows_per_w - 1).wait()

    return gather_k


def _mlp_body(e_ref, w1t_ref, b1_ref, w2_ref, b2_ref, o_ref):
    ht = jnp.dot(w1t_ref[...], e_ref[...],
                 preferred_element_type=jnp.float32)
    ht = jnp.maximum(ht + b1_ref[...], 0.0)
    o_ref[...] = jnp.dot(w2_ref[...], ht,
                         preferred_element_type=jnp.float32) + b2_ref[...]


def kernel(x, tables, W1, b1, W2, b2):
    batch = x.shape[0]
    tlin = jnp.transpose(tables, (0, 2, 1)).reshape(N_FIELDS * D * VOCAB)
    xt = jnp.transpose(x.astype(jnp.int32), (1, 0))

    e3 = _make_sc_gather(batch)(tlin, xt)

    blk = 1024
    w1t = jnp.transpose(W1, (1, 0))
    outT = pl.pallas_call(
        _mlp_body,
        grid=(batch // blk,),
        in_specs=[
            pl.BlockSpec((FD, blk), lambda i: (0, i)),
            pl.BlockSpec((H, FD), lambda i: (0, 0)),
            pl.BlockSpec((H, 1), lambda i: (0, 0)),
            pl.BlockSpec((1, H), lambda i: (0, 0)),
            pl.BlockSpec((1, 1), lambda i: (0, 0)),
        ],
        out_specs=pl.BlockSpec((1, blk), lambda i: (0, i)),
        out_shape=jax.ShapeDtypeStruct((1, batch), jnp.float32),
    )(e3, w1t, b1.reshape(H, 1), W2.reshape(1, H), b2.reshape(1, 1))
    return outT.reshape(batch, 1)


# speedup vs baseline: 1.2372x; 1.0055x over previous
import functools
import jax
import jax.numpy as jnp
from jax import lax
from jax.experimental import pallas as pl
from jax.experimental.pallas import tpu as pltpu
from jax.experimental.pallas import tpu_sc as plsc

N_FIELDS = 26
VOCAB = 100000
D = 32
H = 64
FD = N_FIELDS * D


def _make_sc_gather(batch):
    info = plsc.get_sparse_core_info()
    nc, ns = info.num_cores, info.num_subcores
    nw = nc * ns
    rows_per_w = FD // nw
    mesh = plsc.VectorSubcoreMesh(core_axis_name="c", subcore_axis_name="s")

    @functools.partial(
        pl.kernel,
        mesh=mesh,
        compiler_params=pltpu.CompilerParams(use_tc_tiling_on_sc=False),
        out_type=jax.ShapeDtypeStruct((FD, batch), jnp.float32),
        scratch_types=[
            pltpu.VMEM((2, batch), jnp.int32),
            pltpu.VMEM((2, batch), jnp.int32),
            pltpu.VMEM((2, batch), jnp.float32),
            pltpu.SemaphoreType.DMA,
            pltpu.SemaphoreType.DMA,
        ],
    )
    def gather_k(table_hbm, xt_hbm, out_hbm, xf_v, idx_v, row_v, sg, so):
        wid = lax.axis_index("s") * nc + lax.axis_index("c")
        p0 = wid * rows_per_w
        fstart = jnp.minimum(p0 // D, N_FIELDS - 2)
        pltpu.sync_copy(xt_hbm.at[pl.ds(fstart, 2)], xf_v)

        def build(i):
            p = p0 + i
            frel = p // D - fstart
            base = p * VOCAB

            def addv(j, carry):
                idx_v[i % 2, pl.ds(j * 16, 16)] = (
                    xf_v[frel, pl.ds(j * 16, 16)] + base)
                return carry

            lax.fori_loop(0, batch // 16, addv, 0, unroll=False)

        def gather(i):
            return pltpu.make_async_copy(
                table_hbm.at[idx_v.at[i % 2]], row_v.at[i % 2], sg)

        def writeout(i):
            return pltpu.make_async_copy(
                row_v.at[i % 2], out_hbm.at[p0 + i], so)

        build(0)
        gather(0).start()
        build(1)
        for i in range(rows_per_w):
            if i + 1 < rows_per_w:
                if i >= 1:
                    writeout(i - 1).wait()
                gather(i + 1).start()
                if i + 2 < rows_per_w:
                    build(i + 2)
            gather(i).wait()
            writeout(i).start()
        writeout(rows_per_w - 1).wait()

    return gather_k


def _mlp_body(e_ref, w1t_ref, b1_ref, w2_ref, b2_ref, o_ref):
    ht = jnp.dot(w1t_ref[...], e_ref[...],
                 preferred_element_type=jnp.float32)
    ht = jnp.maximum(ht + b1_ref[...], 0.0)
    o_ref[...] = jnp.dot(w2_ref[...], ht,
                         preferred_element_type=jnp.float32) + b2_ref[...]


def kernel(x, tables, W1, b1, W2, b2):
    batch = x.shape[0]
    tlin = jnp.transpose(tables, (0, 2, 1)).reshape(N_FIELDS * D * VOCAB)
    xt = jnp.transpose(x.astype(jnp.int32), (1, 0))

    e3 = _make_sc_gather(batch)(tlin, xt)

    blk = 1024
    w1t = jnp.transpose(W1, (1, 0))
    outT = pl.pallas_call(
        _mlp_body,
        grid=(batch // blk,),
        in_specs=[
            pl.BlockSpec((FD, blk), lambda i: (0, i)),
            pl.BlockSpec((H, FD), lambda i: (0, 0)),
            pl.BlockSpec((H, 1), lambda i: (0, 0)),
            pl.BlockSpec((1, H), lambda i: (0, 0)),
            pl.BlockSpec((1, 1), lambda i: (0, 0)),
        ],
        out_specs=pl.BlockSpec((1, blk), lambda i: (0, i)),
        out_shape=jax.ShapeDtypeStruct((1, batch), jnp.float32),
    )(e3, w1t, b1.reshape(H, 1), W2.reshape(1, H), b2.reshape(1, 1))
    return outT.reshape(batch, 1)
